# Initial kernel scaffold; baseline (speedup 1.0000x reference)
import jax
import jax.numpy as jnp
from jax.experimental import pallas as pl


def kernel(x, edge_index, W_l1, b_l1, W_r1, W_l2, b_l2, W_r2):
    raise NotImplementedError("write your pallas kernel here")


# speedup vs baseline: 4.2135x; 4.2135x over previous
import jax
import jax.numpy as jnp
from jax import lax
from jax.experimental import pallas as pl
from jax.experimental.pallas import tpu as pltpu
from jax.experimental.pallas import tpu_sc as plsc

_N = 10000
_NPAD = 10240
_E = 160000
_D = 256
_DH = 128
_CHUNK = 128
_NCHUNKS = _E // _CHUNK
_NSUB = 16
_ROWS_PER_SUB = _NPAD // _NSUB
_TRIPS = -(-_NCHUNKS // _NSUB)


def _make_sc_agg(width: int):
    mesh = plsc.VectorSubcoreMesh(core_axis_name="c", subcore_axis_name="s")
    out_type = (
        jax.ShapeDtypeStruct((_NPAD, width), jnp.float32),
        jax.ShapeDtypeStruct((_NPAD, width), jnp.float32),
    )
    scratch = [
        pltpu.VMEM_SHARED((_NPAD, width), jnp.float32),
        pltpu.VMEM((_CHUNK,), jnp.int32),
        pltpu.VMEM((_CHUNK,), jnp.int32),
        pltpu.VMEM((_CHUNK, width), jnp.float32),
        pltpu.SemaphoreType.DMA,
    ]

    def body(x_lo, x_hi, src, dst, out_lo, out_hi, acc, src_v, dst_v,
             rows_v, sem):
        c = lax.axis_index("c")
        t = lax.axis_index("s")
        r0 = t * _ROWS_PER_SUB
        nsplit = _ROWS_PER_SUB // _CHUNK

        zv = jnp.zeros((16,), jnp.float32)

        def zrow(i, _):
            for k in range(width // 16):
                rows_v[i, pl.ds(k * 16, 16)] = zv
            return 0

        lax.fori_loop(0, _CHUNK, zrow, 0)
        for k in range(nsplit):
            pltpu.sync_copy(rows_v, acc.at[pl.ds(r0 + k * _CHUNK, _CHUNK)])
        plsc.subcore_barrier()

        def edge_loop(x_half):
            def step(j, _):
                chunk = t + _NSUB * j

                @pl.when(chunk < _NCHUNKS)
                def _():
                    base = chunk * _CHUNK
                    pltpu.sync_copy(src.at[pl.ds(base, _CHUNK)], src_v)
                    pltpu.sync_copy(dst.at[pl.ds(base, _CHUNK)], dst_v)
                    pltpu.async_copy(x_half.at[src_v], rows_v, sem).wait()
                    pltpu.sync_copy(rows_v, acc.at[dst_v], add=True)
                return 0

            lax.fori_loop(0, _TRIPS, step, 0)

        @pl.when(c == 0)
        def _():
            edge_loop(x_lo)

        @pl.when(c == 1)
        def _():
            edge_loop(x_hi)

        plsc.subcore_barrier()

        def write_out(out_ref):
            for k in range(nsplit):
                sl = pl.ds(r0 + k * _CHUNK, _CHUNK)
                pltpu.sync_copy(acc.at[sl], rows_v)
                pltpu.sync_copy(rows_v, out_ref.at[sl])

        @pl.when(c == 0)
        def _():
            write_out(out_lo)

        @pl.when(c == 1)
        def _():
            write_out(out_hi)

    return pl.kernel(body, out_type=out_type, mesh=mesh,
                     scratch_types=scratch,
                     compiler_params=pltpu.CompilerParams(
                         use_tc_tiling_on_sc=False))


_sc_agg_deg = _make_sc_agg(_DH + 16)
_sc_agg = _make_sc_agg(_DH)

_RB = 1000


def _tc1_body(agg_lo, agg_hi, deg, x, wl, wr, b, h_lo, h_hi):
    inv = 1.0 / jnp.maximum(deg[:, 0:1], 1.0)
    ml = agg_lo[...] * inv
    mh = agg_hi[...] * inv
    h = (jnp.dot(ml, wl[:_DH, :], preferred_element_type=jnp.float32)
         + jnp.dot(mh, wl[_DH:, :], preferred_element_type=jnp.float32)
         + jnp.dot(x[...], wr[...], preferred_element_type=jnp.float32)
         + b[...])
    h = jnp.maximum(h, 0.0)
    h_lo[...] = h[:, :_DH]
    h_hi[...] = h[:, _DH:]


def _tc2_body(agg_lo, agg_hi, deg, h_lo, h_hi, wl, wr, b, out):
    inv = 1.0 / jnp.maximum(deg[:, 0:1], 1.0)
    z = (jnp.dot(agg_lo[...] * inv, wl[:_DH, :], preferred_element_type=jnp.float32)
         + jnp.dot(agg_hi[...] * inv, wl[_DH:, :], preferred_element_type=jnp.float32)
         + jnp.dot(h_lo[...], wr[:_DH, :], preferred_element_type=jnp.float32)
         + jnp.dot(h_hi[...], wr[_DH:, :], preferred_element_type=jnp.float32)
         + b[...])
    zc = z - jnp.max(z, axis=-1, keepdims=True)
    out[...] = zc - jnp.log(jnp.sum(jnp.exp(zc), axis=-1, keepdims=True))


def _row_block(cols):
    return pl.BlockSpec((_RB, cols), lambda i: (i, 0))


def _full_block(rows, cols):
    return pl.BlockSpec((rows, cols), lambda i: (0, 0))


_tc1 = pl.pallas_call(
    _tc1_body,
    grid=(_N // _RB,),
    in_specs=[_row_block(_DH), _row_block(_DH), _row_block(16), _row_block(_D),
              _full_block(_D, _D), _full_block(_D, _D), _full_block(1, _D)],
    out_specs=[_row_block(_DH), _row_block(_DH)],
    out_shape=[jax.ShapeDtypeStruct((_N, _DH), jnp.float32),
               jax.ShapeDtypeStruct((_N, _DH), jnp.float32)],
)

_tc2 = pl.pallas_call(
    _tc2_body,
    grid=(_N // _RB,),
    in_specs=[_row_block(_DH), _row_block(_DH), _row_block(16),
              _row_block(_DH), _row_block(_DH),
              _full_block(_D, _D), _full_block(_D, _D), _full_block(1, _D)],
    out_specs=_row_block(_D),
    out_shape=jax.ShapeDtypeStruct((_N, _D), jnp.float32),
)


def kernel(x, edge_index, W_l1, b_l1, W_r1, W_l2, b_l2, W_r2):
    src = edge_index[0]
    dst = edge_index[1]
    ones16 = jnp.ones((_N, 16), jnp.float32)
    x_lo = jnp.concatenate([x[:, :_DH], ones16], axis=1)
    x_hi = jnp.concatenate([x[:, _DH:], ones16], axis=1)

    agg_lo_a, agg_hi_a = _sc_agg_deg(x_lo, x_hi, src, dst)
    agg_lo = agg_lo_a[:_N, :_DH]
    agg_hi = agg_hi_a[:_N, :_DH]
    deg = agg_lo_a[:_N, _DH:]
    h_lo, h_hi = _tc1(agg_lo, agg_hi, deg, x, W_l1, W_r1, b_l1.reshape(1, _D))
    agg2_lo, agg2_hi = _sc_agg(h_lo, h_hi, src, dst)
    return _tc2(agg2_lo[:_N], agg2_hi[:_N], deg, h_lo, h_hi, W_l2, W_r2,
                b_l2.reshape(1, _D))


# speedup vs baseline: 7.4125x; 1.7592x over previous
import jax
import jax.numpy as jnp
from jax import lax
from jax.experimental import pallas as pl
from jax.experimental.pallas import tpu as pltpu
from jax.experimental.pallas import tpu_sc as plsc

_N = 10000
_NPAD = 10240
_E = 160000
_D = 256
_DH = 128
_CHUNK = 128
_NCHUNKS = _E // _CHUNK
_NSUB = 16
_ROWS_PER_SUB = _NPAD // _NSUB
_TRIPS = -(-_NCHUNKS // _NSUB)


_PAIRS = 39
_FULL = 2 * _PAIRS


def _make_sc_agg(width: int):
    mesh = plsc.VectorSubcoreMesh(core_axis_name="c", subcore_axis_name="s")
    out_type = (
        jax.ShapeDtypeStruct((_NPAD, width), jnp.float32),
        jax.ShapeDtypeStruct((_NPAD, width), jnp.float32),
    )
    scratch = [
        pltpu.VMEM_SHARED((_NPAD, width), jnp.float32),
        pltpu.VMEM((2, _CHUNK), jnp.int32),
        pltpu.VMEM((2, _CHUNK), jnp.int32),
        pltpu.VMEM((_CHUNK, width), jnp.float32),
        pltpu.VMEM((_CHUNK, width), jnp.float32),
        pltpu.SemaphoreType.DMA,
        pltpu.SemaphoreType.DMA,
    ]

    def body(x_lo, x_hi, ec, out_lo, out_hi, acc, idx_a, idx_b,
             rows_a, rows_b, sem_a, sem_b):
        c = lax.axis_index("c")
        t = lax.axis_index("s")
        r0 = t * _ROWS_PER_SUB
        nsplit = _ROWS_PER_SUB // _CHUNK

        zv = jnp.zeros((16,), jnp.float32)

        def zrow(i, _):
            for k in range(width // 16):
                rows_a[i, pl.ds(k * 16, 16)] = zv
            return 0

        lax.fori_loop(0, _CHUNK, zrow, 0)
        for k in range(nsplit):
            pltpu.sync_copy(rows_a, acc.at[pl.ds(r0 + k * _CHUNK, _CHUNK)])
        plsc.subcore_barrier()

        def edge_loop(x_half):
            def gather(idx_v, rows_v, sem):
                return pltpu.make_async_copy(x_half.at[idx_v.at[0]],
                                             rows_v, sem)

            pltpu.sync_copy(ec.at[t], idx_a)
            gather(idx_a, rows_a, sem_a).start()

            def pair(k, _):
                j1 = 2 * k + 1
                pltpu.sync_copy(ec.at[t + _NSUB * j1], idx_b)
                gather(idx_b, rows_b, sem_b).start()
                gather(idx_a, rows_a, sem_a).wait()
                pltpu.sync_copy(rows_a, acc.at[idx_a.at[1]], add=True)

                @pl.when(k < _PAIRS - 1)
                def _():
                    pltpu.sync_copy(ec.at[t + _NSUB * (j1 + 1)], idx_a)
                    gather(idx_a, rows_a, sem_a).start()

                gather(idx_b, rows_b, sem_b).wait()
                pltpu.sync_copy(rows_b, acc.at[idx_b.at[1]], add=True)
                return 0

            lax.fori_loop(0, _PAIRS, pair, 0)

            @pl.when(t < _NCHUNKS - _NSUB * _FULL)
            def _():
                pltpu.sync_copy(ec.at[t + _NSUB * _FULL], idx_a)
                gather(idx_a, rows_a, sem_a).start()
                gather(idx_a, rows_a, sem_a).wait()
                pltpu.sync_copy(rows_a, acc.at[idx_a.at[1]], add=True)

        @pl.when(c == 0)
        def _():
            edge_loop(x_lo)

        @pl.when(c == 1)
        def _():
            edge_loop(x_hi)

        plsc.subcore_barrier()

        def write_out(out_ref):
            for k in range(nsplit):
                sl = pl.ds(r0 + k * _CHUNK, _CHUNK)
                pltpu.sync_copy(acc.at[sl], rows_a)
                pltpu.sync_copy(rows_a, out_ref.at[sl])

        @pl.when(c == 0)
        def _():
            write_out(out_lo)

        @pl.when(c == 1)
        def _():
            write_out(out_hi)

    return pl.kernel(body, out_type=out_type, mesh=mesh,
                     scratch_types=scratch,
                     compiler_params=pltpu.CompilerParams(
                         use_tc_tiling_on_sc=False))


_sc_agg_deg = _make_sc_agg(_DH + 16)
_sc_agg = _make_sc_agg(_DH)

_RB = 1000


def _tc1_body(agg_lo, agg_hi, x, wl, wr, b, h_lo, h_hi):
    inv = 1.0 / jnp.maximum(agg_lo[:, _DH:_DH + 1], 1.0)
    ml = agg_lo[:, :_DH] * inv
    mh = agg_hi[:, :_DH] * inv
    h = (jnp.dot(ml, wl[:_DH, :], preferred_element_type=jnp.float32)
         + jnp.dot(mh, wl[_DH:, :], preferred_element_type=jnp.float32)
         + jnp.dot(x[...], wr[...], preferred_element_type=jnp.float32)
         + b[...])
    h = jnp.maximum(h, 0.0)
    h_lo[...] = h[:, :_DH]
    h_hi[...] = h[:, _DH:]


def _tc2_body(agg_lo, agg_hi, deg144, h_lo, h_hi, wl, wr, b, out):
    inv = 1.0 / jnp.maximum(deg144[:, _DH:_DH + 1], 1.0)
    z = (jnp.dot(agg_lo[...] * inv, wl[:_DH, :], preferred_element_type=jnp.float32)
         + jnp.dot(agg_hi[...] * inv, wl[_DH:, :], preferred_element_type=jnp.float32)
         + jnp.dot(h_lo[...], wr[:_DH, :], preferred_element_type=jnp.float32)
         + jnp.dot(h_hi[...], wr[_DH:, :], preferred_element_type=jnp.float32)
         + b[...])
    zc = z - jnp.max(z, axis=-1, keepdims=True)
    out[...] = zc - jnp.log(jnp.sum(jnp.exp(zc), axis=-1, keepdims=True))


def _row_block(cols):
    return pl.BlockSpec((_RB, cols), lambda i: (i, 0))


def _full_block(rows, cols):
    return pl.BlockSpec((rows, cols), lambda i: (0, 0))


_tc1 = pl.pallas_call(
    _tc1_body,
    grid=(_N // _RB,),
    in_specs=[_row_block(_DH + 16), _row_block(_DH + 16), _row_block(_D),
              _full_block(_D, _D), _full_block(_D, _D), _full_block(1, _D)],
    out_specs=[_row_block(_DH), _row_block(_DH)],
    out_shape=[jax.ShapeDtypeStruct((_N, _DH), jnp.float32),
               jax.ShapeDtypeStruct((_N, _DH), jnp.float32)],
)

_tc2 = pl.pallas_call(
    _tc2_body,
    grid=(_N // _RB,),
    in_specs=[_row_block(_DH), _row_block(_DH), _row_block(_DH + 16),
              _row_block(_DH), _row_block(_DH),
              _full_block(_D, _D), _full_block(_D, _D), _full_block(1, _D)],
    out_specs=_row_block(_D),
    out_shape=jax.ShapeDtypeStruct((_N, _D), jnp.float32),
)


def kernel(x, edge_index, W_l1, b_l1, W_r1, W_l2, b_l2, W_r2):
    ec = edge_index.reshape(2, _NCHUNKS, _CHUNK).transpose(1, 0, 2)
    ones16 = jnp.ones((_N, 16), jnp.float32)
    x_lo = jnp.concatenate([x[:, :_DH], ones16], axis=1)
    x_hi = jnp.concatenate([x[:, _DH:], ones16], axis=1)

    agg_lo_a, agg_hi_a = _sc_agg_deg(x_lo, x_hi, ec)
    h_lo, h_hi = _tc1(agg_lo_a, agg_hi_a, x, W_l1, W_r1, b_l1.reshape(1, _D))
    agg2_lo, agg2_hi = _sc_agg(h_lo, h_hi, ec)
    return _tc2(agg2_lo, agg2_hi, agg_lo_a, h_lo, h_hi, W_l2, W_r2,
                b_l2.reshape(1, _D))
